# Initial kernel scaffold; baseline (speedup 1.0000x reference)
import jax
import jax.numpy as jnp
from jax.experimental import pallas as pl


def kernel(x, edge_index, W1, b1, W2, b2):
    raise NotImplementedError("write your pallas kernel here")


# speedup vs baseline: 10.2328x; 10.2328x over previous
import functools

import jax
import jax.numpy as jnp
from jax import lax
from jax.experimental import pallas as pl
from jax.experimental.pallas import tpu as pltpu
from jax.experimental.pallas import tpu_sc as plsc

N = 10000
E = 320000
D_IN = 128
D_HID = 128
D_OUT = 40
D2 = 48

NC, NS, L = 2, 16, 16
NW = NC * NS
CHUNK = 128
EPW = 10240
E_PAD = NW * EPW
NCHUNK = EPW // CHUNK
NP = 10240
RPT = NP // NS

BLK = 512
GRID = NP // BLK

_MESH = plsc.VectorSubcoreMesh(
    core_axis_name="c", subcore_axis_name="s", num_cores=NC, num_subcores=NS)


@functools.partial(
    pl.kernel,
    out_type=jax.ShapeDtypeStruct((NC, NP, 128), jnp.float32),
    mesh=_MESH,
    scratch_types=[
        pltpu.VMEM((NCHUNK, CHUNK), jnp.int32),
        pltpu.VMEM((CHUNK, 128), jnp.float32),
        pltpu.VMEM_SHARED((NP, 128), jnp.float32),
    ],
    compiler_params=pltpu.CompilerParams(use_tc_tiling_on_sc=False),
)
def _deg_sc(dst_hbm, ones_hbm, zeros_hbm, out_hbm, dst_v, ones_v, acc_sh):
    cid = lax.axis_index("c")
    sid = lax.axis_index("s")
    wid = sid * NC + cid
    pltpu.sync_copy(zeros_hbm, acc_sh.at[pl.ds(sid * RPT, RPT)])
    pltpu.sync_copy(dst_hbm.at[wid], dst_v)
    pltpu.sync_copy(ones_hbm, ones_v)
    plsc.subcore_barrier()

    def body(c, _):
        pltpu.sync_copy(ones_v, acc_sh.at[dst_v.at[c]], add=True)
        return ()

    lax.fori_loop(0, NCHUNK, body, (), unroll=False)
    plsc.subcore_barrier()
    pltpu.sync_copy(acc_sh.at[pl.ds(sid * RPT, RPT)],
                    out_hbm.at[cid, pl.ds(sid * RPT, RPT)])


def _make_agg(D):
    @functools.partial(
        pl.kernel,
        out_type=jax.ShapeDtypeStruct((NC, NP, D), jnp.float32),
        mesh=_MESH,
        scratch_types=[
            pltpu.VMEM((NCHUNK, CHUNK), jnp.int32),
            pltpu.VMEM((NCHUNK, CHUNK), jnp.int32),
            pltpu.VMEM((CHUNK, D), jnp.float32),
            pltpu.VMEM_SHARED((NP, D), jnp.float32),
            pltpu.SemaphoreType.DMA,
        ],
        compiler_params=pltpu.CompilerParams(use_tc_tiling_on_sc=False),
    )
    def agg(hs_hbm, src_hbm, dst_hbm, zeros_hbm, out_hbm,
            src_v, dst_v, rows_v, acc_sh, sem):
        cid = lax.axis_index("c")
        sid = lax.axis_index("s")
        wid = sid * NC + cid
        pltpu.sync_copy(zeros_hbm, acc_sh.at[pl.ds(sid * RPT, RPT)])
        pltpu.sync_copy(src_hbm.at[wid], src_v)
        pltpu.sync_copy(dst_hbm.at[wid], dst_v)
        plsc.subcore_barrier()

        def body(c, _):
            pltpu.async_copy(hs_hbm.at[src_v.at[c]], rows_v, sem).wait()
            pltpu.sync_copy(rows_v, acc_sh.at[dst_v.at[c]], add=True)
            return ()

        lax.fori_loop(0, NCHUNK, body, (), unroll=False)
        plsc.subcore_barrier()
        pltpu.sync_copy(acc_sh.at[pl.ds(sid * RPT, RPT)],
                        out_hbm.at[cid, pl.ds(sid * RPT, RPT)])

    return agg


_agg128 = _make_agg(D_HID)
_agg48 = _make_agg(D2)


def _dinv_of(degp):
    deg = 1.0 + degp[0] + degp[1]
    return lax.rsqrt(deg)[:, 0:1]


def _mm1_body(x_ref, w_ref, degp_ref, out_ref):
    dinv = _dinv_of(degp_ref[...])
    h = jnp.dot(x_ref[...], w_ref[...], preferred_element_type=jnp.float32)
    out_ref[...] = h * dinv


def _mid_body(a_ref, hs1_ref, degp_ref, b1_ref, w2_ref, out_ref):
    a = a_ref[...]
    dinv = _dinv_of(degp_ref[...])
    z = jnp.maximum(dinv * (a[0] + a[1] + hs1_ref[...]) + b1_ref[...], 0.0)
    h2 = jnp.dot(z, w2_ref[...], preferred_element_type=jnp.float32)
    out_ref[...] = h2 * dinv


def _out_body(a_ref, hs2_ref, degp_ref, b2_ref, out_ref):
    a = a_ref[...]
    dinv = _dinv_of(degp_ref[...])
    t = dinv * (a[0] + a[1] + hs2_ref[...]) + b2_ref[...]
    col = lax.broadcasted_iota(jnp.int32, t.shape, 1)
    valid = col < D_OUT
    tm = jnp.where(valid, t, -jnp.inf)
    m = jnp.max(tm, axis=1, keepdims=True)
    e = jnp.where(valid, jnp.exp(t - m), 0.0)
    s = jnp.sum(e, axis=1, keepdims=True)
    out_ref[...] = t - m - jnp.log(s)


def _row_spec(d):
    return pl.BlockSpec((BLK, d), lambda i: (i, 0))


_DEGP_SPEC = pl.BlockSpec((NC, BLK, 128), lambda i: (0, i, 0))
_FULL2 = lambda r, c: pl.BlockSpec((r, c), lambda i: (0, 0))


_mm1_tc = pl.pallas_call(
    _mm1_body,
    grid=(GRID,),
    in_specs=[_row_spec(D_IN), _FULL2(D_IN, D_HID), _DEGP_SPEC],
    out_specs=_row_spec(D_HID),
    out_shape=jax.ShapeDtypeStruct((NP, D_HID), jnp.float32),
)

_mid_tc = pl.pallas_call(
    _mid_body,
    grid=(GRID,),
    in_specs=[pl.BlockSpec((NC, BLK, D_HID), lambda i: (0, i, 0)),
              _row_spec(D_HID), _DEGP_SPEC, _FULL2(1, D_HID),
              _FULL2(D_HID, D2)],
    out_specs=_row_spec(D2),
    out_shape=jax.ShapeDtypeStruct((NP, D2), jnp.float32),
)

_out_tc = pl.pallas_call(
    _out_body,
    grid=(GRID,),
    in_specs=[pl.BlockSpec((NC, BLK, D2), lambda i: (0, i, 0)),
              _row_spec(D2), _DEGP_SPEC, _FULL2(1, D2)],
    out_specs=_row_spec(D2),
    out_shape=jax.ShapeDtypeStruct((NP, D2), jnp.float32),
)


def kernel(x, edge_index, W1, b1, W2, b2):
    src = edge_index[0].astype(jnp.int32)
    dst = edge_index[1].astype(jnp.int32)
    pad = jnp.full((E_PAD - E,), N, jnp.int32)
    src3 = jnp.concatenate([src, pad]).reshape(NW, NCHUNK, CHUNK)
    dst3 = jnp.concatenate([dst, pad]).reshape(NW, NCHUNK, CHUNK)
    x_p = jnp.zeros((NP, D_IN), jnp.float32).at[:N].set(x)
    W2p = jnp.zeros((D_HID, D2), jnp.float32).at[:, :D_OUT].set(W2)
    b2p = jnp.zeros((1, D2), jnp.float32).at[0, :D_OUT].set(b2)
    b1r = b1.reshape(1, D_HID)
    zeros128 = jnp.zeros((RPT, D_HID), jnp.float32)
    zeros48 = jnp.zeros((RPT, D2), jnp.float32)
    ones128 = jnp.ones((CHUNK, 128), jnp.float32)

    degp = _deg_sc(dst3, ones128, zeros128)
    hs1 = _mm1_tc(x_p, W1, degp)
    agg1 = _agg128(hs1, src3, dst3, zeros128)
    hs2 = _mid_tc(agg1, hs1, degp, b1r, W2p)
    agg2 = _agg48(hs2, src3, dst3, zeros48)
    outp = _out_tc(agg2, hs2, degp, b2p)
    return outp[:N, :D_OUT]


# speedup vs baseline: 11.4359x; 1.1176x over previous
import functools

import jax
import jax.numpy as jnp
from jax import lax
from jax.experimental import pallas as pl
from jax.experimental.pallas import tpu as pltpu
from jax.experimental.pallas import tpu_sc as plsc

N = 10000
E = 320000
D_IN = 128
D_HID = 128
D_OUT = 40
D2 = 48

NC, NS, L = 2, 16, 16
NW = NC * NS
CHUNK = 128
EPW = 10240
E_PAD = NW * EPW
NCHUNK = EPW // CHUNK
NP = 10240
RPT = NP // NS

BLK = 512
GRID = NP // BLK

_MESH = plsc.VectorSubcoreMesh(
    core_axis_name="c", subcore_axis_name="s", num_cores=NC, num_subcores=NS)


@functools.partial(
    pl.kernel,
    out_type=jax.ShapeDtypeStruct((NC, NP, 128), jnp.float32),
    mesh=_MESH,
    scratch_types=[
        pltpu.VMEM((NCHUNK, CHUNK), jnp.int32),
        pltpu.VMEM((CHUNK, 128), jnp.float32),
        pltpu.VMEM_SHARED((NP, 128), jnp.float32),
    ],
    compiler_params=pltpu.CompilerParams(use_tc_tiling_on_sc=False),
)
def _deg_sc(dst_hbm, ones_hbm, zeros_hbm, out_hbm, dst_v, ones_v, acc_sh):
    cid = lax.axis_index("c")
    sid = lax.axis_index("s")
    wid = sid * NC + cid
    pltpu.sync_copy(zeros_hbm, acc_sh.at[pl.ds(sid * RPT, RPT)])
    pltpu.sync_copy(dst_hbm.at[wid], dst_v)
    pltpu.sync_copy(ones_hbm, ones_v)
    plsc.subcore_barrier()

    def body(c, _):
        pltpu.sync_copy(ones_v, acc_sh.at[dst_v.at[c]], add=True)
        return ()

    lax.fori_loop(0, NCHUNK, body, (), unroll=False)
    plsc.subcore_barrier()
    pltpu.sync_copy(acc_sh.at[pl.ds(sid * RPT, RPT)],
                    out_hbm.at[cid, pl.ds(sid * RPT, RPT)])


def _make_agg(D):
    @functools.partial(
        pl.kernel,
        out_type=jax.ShapeDtypeStruct((NC, NP, D), jnp.float32),
        mesh=_MESH,
        scratch_types=[
            pltpu.VMEM((CHUNK,), jnp.int32),
            pltpu.VMEM((CHUNK,), jnp.int32),
            pltpu.VMEM((CHUNK,), jnp.int32),
            pltpu.VMEM((CHUNK,), jnp.int32),
            pltpu.VMEM((CHUNK, D), jnp.float32),
            pltpu.VMEM((CHUNK, D), jnp.float32),
            pltpu.VMEM_SHARED((NP, D), jnp.float32),
            pltpu.SemaphoreType.DMA,
            pltpu.SemaphoreType.DMA,
        ],
        compiler_params=pltpu.CompilerParams(use_tc_tiling_on_sc=False),
    )
    def agg(hs_hbm, src_hbm, dst_hbm, zeros_hbm, out_hbm,
            si0, si1, di0, di1, rows0, rows1, acc_sh, sem0, sem1):
        cid = lax.axis_index("c")
        sid = lax.axis_index("s")
        wid = sid * NC + cid
        pltpu.sync_copy(zeros_hbm, acc_sh.at[pl.ds(sid * RPT, RPT)])
        pltpu.sync_copy(src_hbm.at[wid, 0], si0)
        pltpu.sync_copy(dst_hbm.at[wid, 0], di0)
        plsc.subcore_barrier()
        pltpu.async_copy(hs_hbm.at[si0], rows0, sem0)
        pltpu.sync_copy(src_hbm.at[wid, 1], si1)
        pltpu.sync_copy(dst_hbm.at[wid, 1], di1)

        def body(i, _):
            c0 = 2 * i
            c1 = 2 * i + 1
            pltpu.async_copy(hs_hbm.at[si1], rows1, sem1)
            pltpu.make_async_copy(hs_hbm.at[si0], rows0, sem0).wait()
            pltpu.sync_copy(rows0, acc_sh.at[di0], add=True)

            @pl.when(c0 + 2 < NCHUNK)
            def _():
                pltpu.sync_copy(src_hbm.at[wid, c0 + 2], si0)
                pltpu.sync_copy(dst_hbm.at[wid, c0 + 2], di0)
                pltpu.async_copy(hs_hbm.at[si0], rows0, sem0)

            pltpu.make_async_copy(hs_hbm.at[si1], rows1, sem1).wait()
            pltpu.sync_copy(rows1, acc_sh.at[di1], add=True)

            @pl.when(c1 + 2 < NCHUNK)
            def _():
                pltpu.sync_copy(src_hbm.at[wid, c1 + 2], si1)
                pltpu.sync_copy(dst_hbm.at[wid, c1 + 2], di1)

            return ()

        lax.fori_loop(0, NCHUNK // 2, body, (), unroll=False)
        plsc.subcore_barrier()
        pltpu.sync_copy(acc_sh.at[pl.ds(sid * RPT, RPT)],
                        out_hbm.at[cid, pl.ds(sid * RPT, RPT)])

    return agg


_agg128 = _make_agg(D_HID)
_agg48 = _make_agg(D2)


def _dinv_of(degp):
    deg = 1.0 + degp[0] + degp[1]
    return lax.rsqrt(deg)[:, 0:1]


def _mm1_body(x_ref, w_ref, degp_ref, out_ref):
    dinv = _dinv_of(degp_ref[...])
    h = jnp.dot(x_ref[...], w_ref[...], preferred_element_type=jnp.float32)
    out_ref[...] = h * dinv


def _mid_body(a_ref, hs1_ref, degp_ref, b1_ref, w2_ref, out_ref):
    a = a_ref[...]
    dinv = _dinv_of(degp_ref[...])
    z = jnp.maximum(dinv * (a[0] + a[1] + hs1_ref[...]) + b1_ref[...], 0.0)
    h2 = jnp.dot(z, w2_ref[...], preferred_element_type=jnp.float32)
    out_ref[...] = h2 * dinv


def _out_body(a_ref, hs2_ref, degp_ref, b2_ref, out_ref):
    a = a_ref[...]
    dinv = _dinv_of(degp_ref[...])
    t = dinv * (a[0] + a[1] + hs2_ref[...]) + b2_ref[...]
    col = lax.broadcasted_iota(jnp.int32, t.shape, 1)
    valid = col < D_OUT
    tm = jnp.where(valid, t, -jnp.inf)
    m = jnp.max(tm, axis=1, keepdims=True)
    e = jnp.where(valid, jnp.exp(t - m), 0.0)
    s = jnp.sum(e, axis=1, keepdims=True)
    out_ref[...] = t - m - jnp.log(s)


def _row_spec(d):
    return pl.BlockSpec((BLK, d), lambda i: (i, 0))


_DEGP_SPEC = pl.BlockSpec((NC, BLK, 128), lambda i: (0, i, 0))
_FULL2 = lambda r, c: pl.BlockSpec((r, c), lambda i: (0, 0))


_mm1_tc = pl.pallas_call(
    _mm1_body,
    grid=(GRID,),
    in_specs=[_row_spec(D_IN), _FULL2(D_IN, D_HID), _DEGP_SPEC],
    out_specs=_row_spec(D_HID),
    out_shape=jax.ShapeDtypeStruct((NP, D_HID), jnp.float32),
)

_mid_tc = pl.pallas_call(
    _mid_body,
    grid=(GRID,),
    in_specs=[pl.BlockSpec((NC, BLK, D_HID), lambda i: (0, i, 0)),
              _row_spec(D_HID), _DEGP_SPEC, _FULL2(1, D_HID),
              _FULL2(D_HID, D2)],
    out_specs=_row_spec(D2),
    out_shape=jax.ShapeDtypeStruct((NP, D2), jnp.float32),
)

_out_tc = pl.pallas_call(
    _out_body,
    grid=(GRID,),
    in_specs=[pl.BlockSpec((NC, BLK, D2), lambda i: (0, i, 0)),
              _row_spec(D2), _DEGP_SPEC, _FULL2(1, D2)],
    out_specs=_row_spec(D2),
    out_shape=jax.ShapeDtypeStruct((NP, D2), jnp.float32),
)


def kernel(x, edge_index, W1, b1, W2, b2):
    src = edge_index[0].astype(jnp.int32)
    dst = edge_index[1].astype(jnp.int32)
    pad = jnp.full((E_PAD - E,), N, jnp.int32)
    src3 = jnp.concatenate([src, pad]).reshape(NW, NCHUNK, CHUNK)
    dst3 = jnp.concatenate([dst, pad]).reshape(NW, NCHUNK, CHUNK)
    x_p = jnp.zeros((NP, D_IN), jnp.float32).at[:N].set(x)
    W2p = jnp.zeros((D_HID, D2), jnp.float32).at[:, :D_OUT].set(W2)
    b2p = jnp.zeros((1, D2), jnp.float32).at[0, :D_OUT].set(b2)
    b1r = b1.reshape(1, D_HID)
    zeros128 = jnp.zeros((RPT, D_HID), jnp.float32)
    zeros48 = jnp.zeros((RPT, D2), jnp.float32)
    ones128 = jnp.ones((CHUNK, 128), jnp.float32)

    degp = _deg_sc(dst3, ones128, zeros128)
    hs1 = _mm1_tc(x_p, W1, degp)
    agg1 = _agg128(hs1, src3, dst3, zeros128)
    hs2 = _mid_tc(agg1, hs1, degp, b1r, W2p)
    agg2 = _agg48(hs2, src3, dst3, zeros48)
    outp = _out_tc(agg2, hs2, degp, b2p)
    return outp[:N, :D_OUT]
